# baseline (speedup 1.0000x reference)
import functools
import jax
import jax.numpy as jnp
from jax import lax
from jax.experimental import pallas as pl
from jax.experimental.pallas import tpu as pltpu
from jax.experimental.pallas import tpu_sc as plsc

BATCH = 16384
HIST = 200
EMB = 128
N_ROWS = BATCH * HIST
NW = 32
ROWS_PER_W = N_ROWS // NW
BUF_ROWS = 640
N_CHUNKS = ROWS_PER_W // BUF_ROWS
WINDOW = 6

_mesh = plsc.VectorSubcoreMesh(core_axis_name="c", subcore_axis_name="s")


@functools.partial(
    pl.kernel,
    out_type=jax.ShapeDtypeStruct((N_ROWS, EMB), jnp.float32),
    mesh=_mesh,
    scratch_types=[
        pltpu.VMEM((BUF_ROWS, EMB), jnp.float32),
        pltpu.SemaphoreType.DMA,
    ],
)
def _sc_broadcast(table_hbm, out_hbm, buf, sem):
    wid = lax.axis_index("s") * 2 + lax.axis_index("c")
    base = wid * ROWS_PER_W

    pltpu.sync_copy(table_hbm, buf.at[pl.ds(0, 1)])
    regs = [buf[0, pl.ds(16 * j, 16)] for j in range(EMB // 16)]

    def fill(r, _):
        for j in range(EMB // 16):
            buf[r, pl.ds(16 * j, 16)] = regs[j]
        return 0

    lax.fori_loop(1, BUF_ROWS, fill, 0)

    def copy(i):
        return pltpu.make_async_copy(
            buf, out_hbm.at[pl.ds(base + i * BUF_ROWS, BUF_ROWS)], sem
        )

    def body(i, _):
        copy(i).start()

        @pl.when(i >= WINDOW)
        def _():
            copy(i - WINDOW).wait()

        return 0

    lax.fori_loop(0, N_CHUNKS, body, 0)

    def drain(i, _):
        copy(N_CHUNKS - WINDOW + i).wait()
        return 0

    lax.fori_loop(0, WINDOW, drain, 0)


def kernel(indices, table):
    del indices
    out = _sc_broadcast(table)
    return out.reshape(BATCH, HIST, EMB)


# speedup vs baseline: 1.0302x; 1.0302x over previous
import functools
import jax
import jax.numpy as jnp
from jax import lax
from jax.experimental import pallas as pl
from jax.experimental.pallas import tpu as pltpu
from jax.experimental.pallas import tpu_sc as plsc

BATCH = 16384
HIST = 200
EMB = 128
N_ROWS = BATCH * HIST
TC_ROWS = N_ROWS // 2
SC_ROWS = N_ROWS - TC_ROWS
NW = 32
ROWS_PER_W = SC_ROWS // NW
BUF_ROWS = 400
N_CHUNKS = ROWS_PER_W // BUF_ROWS
WINDOW = 6
TC_BLOCK = 12800
TC_GRID = TC_ROWS // TC_BLOCK

_mesh = plsc.VectorSubcoreMesh(core_axis_name="c", subcore_axis_name="s")


@functools.partial(
    pl.kernel,
    out_type=jax.ShapeDtypeStruct((N_ROWS, EMB), jnp.float32),
    mesh=_mesh,
    scratch_types=[
        pltpu.VMEM((BUF_ROWS, EMB), jnp.float32),
        pltpu.SemaphoreType.DMA,
    ],
)
def _sc_broadcast(table_hbm, out_hbm, buf, sem):
    wid = lax.axis_index("s") * 2 + lax.axis_index("c")
    base = TC_ROWS + wid * ROWS_PER_W

    pltpu.sync_copy(table_hbm, buf.at[pl.ds(0, 1)])
    regs = [buf[0, pl.ds(16 * j, 16)] for j in range(EMB // 16)]

    def fill(r, _):
        for j in range(EMB // 16):
            buf[r, pl.ds(16 * j, 16)] = regs[j]
        return 0

    lax.fori_loop(1, BUF_ROWS, fill, 0)

    def copy(i):
        return pltpu.make_async_copy(
            buf, out_hbm.at[pl.ds(base + i * BUF_ROWS, BUF_ROWS)], sem
        )

    def body(i, _):
        copy(i).start()

        @pl.when(i >= WINDOW)
        def _():
            copy(i - WINDOW).wait()

        return 0

    lax.fori_loop(0, N_CHUNKS, body, 0)

    def drain(i, _):
        copy(N_CHUNKS - WINDOW + i).wait()
        return 0

    lax.fori_loop(0, WINDOW, drain, 0)


def _tc_body(table_ref, prev_ref, out_ref):
    del prev_ref
    row = table_ref[0, :]
    out_ref[...] = jnp.broadcast_to(row[None, :], out_ref.shape)


def kernel(indices, table):
    del indices
    sc_part = _sc_broadcast(table)
    out = pl.pallas_call(
        _tc_body,
        grid=(TC_GRID,),
        in_specs=[
            pl.BlockSpec((1, EMB), lambda i: (0, 0)),
            pl.BlockSpec(memory_space=pl.ANY),
        ],
        out_specs=pl.BlockSpec((TC_BLOCK, EMB), lambda i: (i, 0)),
        out_shape=jax.ShapeDtypeStruct((N_ROWS, EMB), jnp.float32),
        input_output_aliases={1: 0},
    )(table, sc_part)
    return out.reshape(BATCH, HIST, EMB)


# speedup vs baseline: 1.0433x; 1.0127x over previous
import functools
import jax
import jax.numpy as jnp
from jax import lax
from jax.experimental import pallas as pl
from jax.experimental.pallas import tpu as pltpu
from jax.experimental.pallas import tpu_sc as plsc

BATCH = 16384
HIST = 200
EMB = 128
N_ROWS = BATCH * HIST
NW = 32
ROWS_PER_W = N_ROWS // NW
BUF_ROWS = 400
SH_ROWS = 4000
SP_CHUNKS = 8
ST_ROWS = ROWS_PER_W - SP_CHUNKS * SH_ROWS
ST_CHUNKS = ST_ROWS // BUF_ROWS
WINDOW = 6

_mesh = plsc.VectorSubcoreMesh(core_axis_name="c", subcore_axis_name="s")


@functools.partial(
    pl.kernel,
    out_type=jax.ShapeDtypeStruct((N_ROWS, EMB), jnp.float32),
    mesh=_mesh,
    scratch_types=[
        pltpu.VMEM((BUF_ROWS, EMB), jnp.float32),
        pltpu.VMEM_SHARED((SH_ROWS, EMB), jnp.float32),
        pltpu.SemaphoreType.DMA,
        pltpu.SemaphoreType.DMA,
    ],
)
def _sc_broadcast(table_hbm, out_hbm, buf, shared, sem, sem_sp):
    sid = lax.axis_index("s")
    wid = sid * 2 + lax.axis_index("c")
    base = wid * ROWS_PER_W

    pltpu.sync_copy(table_hbm, buf.at[pl.ds(0, 1)])
    regs = [buf[0, pl.ds(16 * j, 16)] for j in range(EMB // 16)]

    def fill(r, _):
        for j in range(EMB // 16):
            buf[r, pl.ds(16 * j, 16)] = regs[j]
        return 0

    lax.fori_loop(1, BUF_ROWS, fill, 0)

    @pl.when(sid == 0)
    def _():
        for k in range(SH_ROWS // BUF_ROWS):
            pltpu.sync_copy(buf, shared.at[pl.ds(k * BUF_ROWS, BUF_ROWS)])

    plsc.subcore_barrier()

    def spcopy(i):
        return pltpu.make_async_copy(
            shared, out_hbm.at[pl.ds(base + i * SH_ROWS, SH_ROWS)], sem_sp
        )

    for i in range(SP_CHUNKS):
        spcopy(i).start()

    st_base = base + SP_CHUNKS * SH_ROWS

    def copy(i):
        return pltpu.make_async_copy(
            buf, out_hbm.at[pl.ds(st_base + i * BUF_ROWS, BUF_ROWS)], sem
        )

    def body(i, _):
        copy(i).start()

        @pl.when(i >= WINDOW)
        def _():
            copy(i - WINDOW).wait()

        return 0

    lax.fori_loop(0, ST_CHUNKS, body, 0)

    def drain(i, _):
        copy(ST_CHUNKS - WINDOW + i).wait()
        return 0

    lax.fori_loop(0, WINDOW, drain, 0)

    for i in range(SP_CHUNKS):
        spcopy(i).wait()


def kernel(indices, table):
    del indices
    out = _sc_broadcast(table)
    return out.reshape(BATCH, HIST, EMB)
